# baseline (speedup 1.0000x reference)
import functools

import jax
import jax.numpy as jnp
from jax import lax
from jax.experimental import pallas as pl
from jax.experimental.pallas import tpu as pltpu
from jax.experimental.pallas import tpu_sc as plsc

VOCAB = 1000000
BATCH = 16384
EMBED_DIM = 32
L = 16

_NC = 2
_NS = 16
NW = _NC * _NS
B_PER_W = BATCH // NW
D_CHUNK = 8


def _body(ex_hbm, tgt_hbm, ctx_hbm, out_hbm,
          ex_v, idx_t_v, idx_c_v, gath_t_v, gath_c_v, out_v,
          sem_t, sem_c):
    wid = lax.axis_index("s") * _NC + lax.axis_index("c")
    base = wid * B_PER_W

    pltpu.sync_copy(ex_hbm.at[pl.ds(base * 3, B_PER_W * 3)], ex_v)

    lanes = lax.iota(jnp.int32, L)

    def extract(k, _):
        flat = (lanes + k * L) * 3
        idx_t_v[pl.ds(k * L, L)] = plsc.load_gather(ex_v, [flat])
        idx_c_v[pl.ds(k * L, L)] = plsc.load_gather(ex_v, [flat + 1])
        return _

    lax.fori_loop(0, B_PER_W // L, extract, None)

    for c0 in range(0, EMBED_DIM, D_CHUNK):
        cps = []
        for d in range(c0, c0 + D_CHUNK):
            cps.append(pltpu.async_copy(
                tgt_hbm.at[d].at[idx_t_v], gath_t_v.at[d], sem_t))
            cps.append(pltpu.async_copy(
                ctx_hbm.at[d].at[idx_c_v], gath_c_v.at[d], sem_c))
        for cp in cps:
            cp.wait()

    def compute(k, _):
        s = pl.ds(k * L, L)
        acc = gath_t_v[0, s] * gath_c_v[0, s]
        for d in range(1, EMBED_DIM):
            acc = acc + gath_t_v[d, s] * gath_c_v[d, s]
        out_v[s] = 1.0 / (1.0 + jnp.exp(-acc))
        return _

    lax.fori_loop(0, B_PER_W // L, compute, None)

    pltpu.sync_copy(out_v, out_hbm.at[pl.ds(base, B_PER_W)])


def kernel(examples, target_embeddings, context_embeddings):
    mesh = plsc.VectorSubcoreMesh(core_axis_name="c", subcore_axis_name="s")
    k = functools.partial(
        pl.kernel,
        mesh=mesh,
        compiler_params=pltpu.CompilerParams(
            needs_layout_passes=False,
            use_tc_tiling_on_sc=False,
        ),
        out_type=jax.ShapeDtypeStruct((BATCH,), jnp.float32),
        scratch_types=[
            pltpu.VMEM((B_PER_W * 3,), jnp.int32),
            pltpu.VMEM((B_PER_W,), jnp.int32),
            pltpu.VMEM((B_PER_W,), jnp.int32),
            pltpu.VMEM((EMBED_DIM, B_PER_W), jnp.float32),
            pltpu.VMEM((EMBED_DIM, B_PER_W), jnp.float32),
            pltpu.VMEM((B_PER_W,), jnp.float32),
            pltpu.SemaphoreType.DMA,
            pltpu.SemaphoreType.DMA,
        ],
    )(_body)
    return k(
        examples.reshape(-1),
        target_embeddings.T,
        context_embeddings.T,
    )


# speedup vs baseline: 20.7426x; 20.7426x over previous
import functools

import jax
import jax.numpy as jnp
from jax import lax
from jax.experimental import pallas as pl
from jax.experimental.pallas import tpu as pltpu
from jax.experimental.pallas import tpu_sc as plsc

VOCAB = 1000000
BATCH = 16384
EMBED_DIM = 32
L = 16

_NC = 2
_NS = 16
NW = _NC * _NS
B_PER_W = BATCH // NW
NBUF = 4


def _body(ex_hbm, tgt_hbm, ctx_hbm, out_hbm,
          ex_v, id_t_v, id_c_v, slab_t_v, slab_c_v, part_v, out_v,
          sems_t, sems_c):
    wid = lax.axis_index("s") * _NC + lax.axis_index("c")
    base = wid * B_PER_W

    pltpu.sync_copy(ex_hbm.at[pl.ds(base * 3, B_PER_W * 3)], ex_v)

    lanes = lax.iota(jnp.int32, L)

    def extract(k, _):
        flat = (lanes + k * L) * 3
        id_t_v[pl.ds(k * L, L)] = plsc.load_gather(ex_v, [flat])
        id_c_v[pl.ds(k * L, L)] = plsc.load_gather(ex_v, [flat + 1])
        return _

    lax.fori_loop(0, B_PER_W // L, extract, None)

    def fetch(e, buf):
        idt = id_t_v[pl.ds(e, L)][0]
        idc = id_c_v[pl.ds(e, L)][0]
        bt = pl.multiple_of((idt >> 7) * 128, 128)
        bc = pl.multiple_of((idc >> 7) * 128, 128)
        ct = pltpu.async_copy(
            tgt_hbm.at[:, pl.ds(bt, 128)], slab_t_v.at[buf], sems_t.at[buf])
        cc = pltpu.async_copy(
            ctx_hbm.at[:, pl.ds(bc, 128)], slab_c_v.at[buf], sems_c.at[buf])
        return ct, cc

    for e in range(NBUF):
        fetch(e, e)

    d_lo = lax.iota(jnp.int32, L)
    d_hi = d_lo + L
    bufv = jnp.zeros((L,), jnp.int32)

    def step(e, _):
        buf = lax.rem(e, NBUF)
        pltpu.make_async_copy(
            tgt_hbm.at[:, pl.ds(0, 128)], slab_t_v.at[buf], sems_t.at[buf]
        ).wait()
        pltpu.make_async_copy(
            ctx_hbm.at[:, pl.ds(0, 128)], slab_c_v.at[buf], sems_c.at[buf]
        ).wait()

        lt = jnp.full((L,), id_t_v[pl.ds(e, L)][0] & 127, jnp.int32)
        lc = jnp.full((L,), id_c_v[pl.ds(e, L)][0] & 127, jnp.int32)
        bv = bufv + buf
        t1 = plsc.load_gather(slab_t_v, [bv, d_lo, lt])
        t2 = plsc.load_gather(slab_t_v, [bv, d_hi, lt])
        c1 = plsc.load_gather(slab_c_v, [bv, d_lo, lc])
        c2 = plsc.load_gather(slab_c_v, [bv, d_hi, lc])
        part_v[e, :] = t1 * c1 + t2 * c2

        nxt = jnp.minimum(e + NBUF, B_PER_W - 1)
        fetch(nxt, buf)
        return _

    lax.fori_loop(0, B_PER_W, step, None)

    for b in range(NBUF):
        pltpu.make_async_copy(
            tgt_hbm.at[:, pl.ds(0, 128)], slab_t_v.at[b], sems_t.at[b]
        ).wait()
        pltpu.make_async_copy(
            ctx_hbm.at[:, pl.ds(0, 128)], slab_c_v.at[b], sems_c.at[b]
        ).wait()

    def reduce(k, _):
        rows = lanes + k * L
        acc = plsc.load_gather(part_v, [rows, jnp.zeros((L,), jnp.int32)])
        for j in range(1, L):
            acc = acc + plsc.load_gather(
                part_v, [rows, jnp.full((L,), j, jnp.int32)])
        out_v[pl.ds(k * L, L)] = 1.0 / (1.0 + jnp.exp(-acc))
        return _

    lax.fori_loop(0, B_PER_W // L, reduce, None)

    pltpu.sync_copy(out_v, out_hbm.at[pl.ds(base, B_PER_W)])


def kernel(examples, target_embeddings, context_embeddings):
    mesh = plsc.VectorSubcoreMesh(core_axis_name="c", subcore_axis_name="s")
    k = functools.partial(
        pl.kernel,
        mesh=mesh,
        compiler_params=pltpu.CompilerParams(
            needs_layout_passes=False,
        ),
        out_type=jax.ShapeDtypeStruct((BATCH,), jnp.float32),
        scratch_types=[
            pltpu.VMEM((B_PER_W * 3,), jnp.int32),
            pltpu.VMEM((B_PER_W + L,), jnp.int32),
            pltpu.VMEM((B_PER_W + L,), jnp.int32),
            pltpu.VMEM((NBUF, EMBED_DIM, 128), jnp.float32),
            pltpu.VMEM((NBUF, EMBED_DIM, 128), jnp.float32),
            pltpu.VMEM((B_PER_W, L), jnp.float32),
            pltpu.VMEM((B_PER_W,), jnp.float32),
            pltpu.SemaphoreType.DMA((NBUF,)),
            pltpu.SemaphoreType.DMA((NBUF,)),
        ],
    )(_body)
    return k(
        examples.reshape(-1),
        target_embeddings.T,
        context_embeddings.T,
    )


# speedup vs baseline: 20.9632x; 1.0106x over previous
import functools

import jax
import jax.numpy as jnp
from jax import lax
from jax.experimental import pallas as pl
from jax.experimental.pallas import tpu as pltpu
from jax.experimental.pallas import tpu_sc as plsc

VOCAB = 1000000
BATCH = 16384
EMBED_DIM = 32
L = 16

_NC = 2
_NS = 16
NW = _NC * _NS
B_PER_W = BATCH // NW
NBUF = 6


def _body(ex_hbm, tgt_hbm, ctx_hbm, out_hbm,
          ex_v, id_t_v, id_c_v, slab_t_v, slab_c_v, part_v, out_v,
          sems_t, sems_c):
    wid = lax.axis_index("s") * _NC + lax.axis_index("c")
    base = wid * B_PER_W

    pltpu.sync_copy(ex_hbm.at[pl.ds(base * 3, B_PER_W * 3)], ex_v)

    lanes = lax.iota(jnp.int32, L)

    def extract(k, _):
        flat = (lanes + k * L) * 3
        id_t_v[pl.ds(k * L, L)] = plsc.load_gather(ex_v, [flat])
        id_c_v[pl.ds(k * L, L)] = plsc.load_gather(ex_v, [flat + 1])
        return _

    lax.fori_loop(0, B_PER_W // L, extract, None)

    def fetch(e, buf):
        idt = id_t_v[pl.ds(e, L)][0]
        idc = id_c_v[pl.ds(e, L)][0]
        bt = pl.multiple_of((idt >> 7) * 128, 128)
        bc = pl.multiple_of((idc >> 7) * 128, 128)
        ct = pltpu.async_copy(
            tgt_hbm.at[:, pl.ds(bt, 128)], slab_t_v.at[buf], sems_t.at[buf])
        cc = pltpu.async_copy(
            ctx_hbm.at[:, pl.ds(bc, 128)], slab_c_v.at[buf], sems_c.at[buf])
        return ct, cc

    for e in range(NBUF):
        fetch(e, e)

    d_lo = lax.iota(jnp.int32, L)
    d_hi = d_lo + L
    bufv = jnp.zeros((L,), jnp.int32)

    def step(e, _):
        buf = lax.rem(e, NBUF)
        pltpu.make_async_copy(
            tgt_hbm.at[:, pl.ds(0, 128)], slab_t_v.at[buf], sems_t.at[buf]
        ).wait()
        pltpu.make_async_copy(
            ctx_hbm.at[:, pl.ds(0, 128)], slab_c_v.at[buf], sems_c.at[buf]
        ).wait()

        lt = jnp.full((L,), id_t_v[pl.ds(e, L)][0] & 127, jnp.int32)
        lc = jnp.full((L,), id_c_v[pl.ds(e, L)][0] & 127, jnp.int32)
        bv = bufv + buf
        t1 = plsc.load_gather(slab_t_v, [bv, d_lo, lt])
        t2 = plsc.load_gather(slab_t_v, [bv, d_hi, lt])
        c1 = plsc.load_gather(slab_c_v, [bv, d_lo, lc])
        c2 = plsc.load_gather(slab_c_v, [bv, d_hi, lc])
        part_v[e, :] = t1 * c1 + t2 * c2

        nxt = jnp.minimum(e + NBUF, B_PER_W - 1)
        fetch(nxt, buf)
        return _

    lax.fori_loop(0, B_PER_W, step, None)

    for b in range(NBUF):
        pltpu.make_async_copy(
            tgt_hbm.at[:, pl.ds(0, 128)], slab_t_v.at[b], sems_t.at[b]
        ).wait()
        pltpu.make_async_copy(
            ctx_hbm.at[:, pl.ds(0, 128)], slab_c_v.at[b], sems_c.at[b]
        ).wait()

    def reduce(k, _):
        rows = lanes + k * L
        acc = plsc.load_gather(part_v, [rows, jnp.zeros((L,), jnp.int32)])
        for j in range(1, L):
            acc = acc + plsc.load_gather(
                part_v, [rows, jnp.full((L,), j, jnp.int32)])
        out_v[pl.ds(k * L, L)] = 1.0 / (1.0 + jnp.exp(-acc))
        return _

    lax.fori_loop(0, B_PER_W // L, reduce, None)

    pltpu.sync_copy(out_v, out_hbm.at[pl.ds(base, B_PER_W)])


def kernel(examples, target_embeddings, context_embeddings):
    mesh = plsc.VectorSubcoreMesh(core_axis_name="c", subcore_axis_name="s")
    k = functools.partial(
        pl.kernel,
        mesh=mesh,
        compiler_params=pltpu.CompilerParams(
            needs_layout_passes=False,
        ),
        out_type=jax.ShapeDtypeStruct((BATCH,), jnp.float32),
        scratch_types=[
            pltpu.VMEM((B_PER_W * 3,), jnp.int32),
            pltpu.VMEM((B_PER_W + L,), jnp.int32),
            pltpu.VMEM((B_PER_W + L,), jnp.int32),
            pltpu.VMEM((NBUF, EMBED_DIM, 128), jnp.float32),
            pltpu.VMEM((NBUF, EMBED_DIM, 128), jnp.float32),
            pltpu.VMEM((B_PER_W, L), jnp.float32),
            pltpu.VMEM((B_PER_W,), jnp.float32),
            pltpu.SemaphoreType.DMA((NBUF,)),
            pltpu.SemaphoreType.DMA((NBUF,)),
        ],
    )(_body)
    return k(
        examples.reshape(-1),
        target_embeddings.T,
        context_embeddings.T,
    )
